# Initial kernel scaffold; baseline (speedup 1.0000x reference)
import jax
import jax.numpy as jnp
from jax.experimental import pallas as pl


def kernel(q, action_mask):
    raise NotImplementedError("write your pallas kernel here")


# speedup vs baseline: 6.6416x; 6.6416x over previous
import functools

import jax
import jax.numpy as jnp
from jax import lax
from jax.experimental import pallas as pl
from jax.experimental.pallas import tpu as pltpu

ALPHA = 0.2
ACT_DIM = 100000
BATCH = 128
MASK_LEN = 5000
MPAD = 5120
ROWS_PER_STEP = 8

_K0 = 0
_K1 = 42


def _rotl(x, r):
    return lax.shift_left(x, jnp.uint32(r)) | lax.shift_right_logical(
        x, jnp.uint32(32 - r))


def _threefry_bits(p):
    ks0 = jnp.uint32(_K0)
    ks1 = jnp.uint32(_K1)
    ks2 = ks0 ^ ks1 ^ jnp.uint32(0x1BD11BDA)
    rot1 = (13, 15, 26, 6)
    rot2 = (17, 29, 16, 24)
    x0 = jnp.zeros_like(p) + ks0
    x1 = p + ks1

    def rnds(x0, x1, rots):
        for r in rots:
            x0 = x0 + x1
            x1 = _rotl(x1, r)
            x1 = x1 ^ x0
        return x0, x1

    x0, x1 = rnds(x0, x1, rot1)
    x0 = x0 + ks1
    x1 = x1 + ks2 + jnp.uint32(1)
    x0, x1 = rnds(x0, x1, rot2)
    x0 = x0 + ks2
    x1 = x1 + ks0 + jnp.uint32(2)
    x0, x1 = rnds(x0, x1, rot1)
    x0 = x0 + ks0
    x1 = x1 + ks1 + jnp.uint32(3)
    x0, x1 = rnds(x0, x1, rot2)
    x0 = x0 + ks1
    x1 = x1 + ks2 + jnp.uint32(4)
    x0, x1 = rnds(x0, x1, rot1)
    x0 = x0 + ks2
    x1 = x1 + ks0 + jnp.uint32(5)
    return x0 ^ x1


def _gumbel_from_bits(bits):
    float_bits = lax.shift_right_logical(bits, jnp.uint32(9)) | jnp.uint32(
        0x3F800000)
    floats = lax.bitcast_convert_type(float_bits, jnp.float32) - jnp.float32(1.0)
    tiny = jnp.float32(jnp.finfo(jnp.float32).tiny)
    u = lax.max(tiny, floats * (jnp.float32(1.0) - tiny) + tiny)
    return -jnp.log(-jnp.log(u))


def _select_body(qg_ref, idx_ref, flg_ref, act_ref, logp_ref):
    step = pl.program_id(0)
    qg = qg_ref[...]
    idx = idx_ref[...]
    flg = flg_ref[...]

    kio = lax.broadcasted_iota(jnp.int32, (1, MPAD), 1)
    valid = (kio < MASK_LEN) & (flg == kio)

    qs = qg * jnp.float32(1.0 / ALPHA)
    neg = jnp.float32(-jnp.inf)
    qs_v = jnp.where(valid, qs, neg)

    row = (step * ROWS_PER_STEP
           + lax.broadcasted_iota(jnp.int32, (ROWS_PER_STEP, MPAD), 0))
    p = (row * ACT_DIM + idx).astype(jnp.uint32)
    g = _gumbel_from_bits(_threefry_bits(p))

    m = jnp.max(qs_v, axis=1, keepdims=True)
    e = jnp.exp(qs_v - m)
    s = jnp.sum(jnp.where(valid, e, jnp.float32(0.0)), axis=1, keepdims=True)

    z = jnp.where(valid, qs + g, neg)
    zmax = jnp.max(z, axis=1, keepdims=True)
    big = jnp.int32(2**30)
    kstar = jnp.min(jnp.where(z >= zmax, kio, big), axis=1, keepdims=True)

    hit = kio == kstar
    act = jnp.max(jnp.where(hit, idx, jnp.int32(0)), axis=1, keepdims=True)
    esel = jnp.max(jnp.where(hit, e, jnp.float32(0.0)), axis=1, keepdims=True)

    act_ref[...] = act
    logp_ref[...] = esel / s


def _select(qg, idxp, flags):
    grid = (BATCH // ROWS_PER_STEP,)
    return pl.pallas_call(
        _select_body,
        grid=grid,
        in_specs=[
            pl.BlockSpec((ROWS_PER_STEP, MPAD), lambda i: (i, 0)),
            pl.BlockSpec((1, MPAD), lambda i: (0, 0)),
            pl.BlockSpec((1, MPAD), lambda i: (0, 0)),
        ],
        out_specs=[
            pl.BlockSpec((ROWS_PER_STEP, 1), lambda i: (i, 0)),
            pl.BlockSpec((ROWS_PER_STEP, 1), lambda i: (i, 0)),
        ],
        out_shape=[
            jax.ShapeDtypeStruct((BATCH, 1), jnp.int32),
            jax.ShapeDtypeStruct((BATCH, 1), jnp.float32),
        ],
    )(qg, idxp, flags)


def kernel(q, action_mask):
    idx = action_mask.astype(jnp.int32)
    idxp = jnp.concatenate(
        [idx, jnp.full((MPAD - MASK_LEN,), ACT_DIM, jnp.int32)])

    flags_tbl = jnp.zeros((ACT_DIM + 8,), jnp.int32).at[idxp].set(
        jnp.arange(MPAD, dtype=jnp.int32), mode="drop")
    flags = flags_tbl[idxp]
    qg = jnp.take(q, jnp.where(idxp < ACT_DIM, idxp, 0), axis=1)

    idx_safe = jnp.where(idxp < ACT_DIM, idxp, 0)
    act, logp = _select(qg, idx_safe[None, :], flags[None, :])
    return act, logp
